# baseline (speedup 1.0000x reference)
import functools

import jax
import jax.numpy as jnp
from jax.experimental import pallas as pl
from jax.experimental.pallas import tpu as pltpu


def _body(ip_ref, h_ref, hs_ref, skip_ref, br_ref, ti_ref, fi_ref,
          out_ip_ref, out_h_ref, acc_h_ref, pbf_ref, *, block_i: int):
    k = pl.program_id(0)
    nk = pl.num_programs(0)
    bi = block_i
    n = skip_ref.shape[1]
    dn = (((1,), (0,)), ((), ()))
    hp = jax.lax.Precision.HIGHEST

    @pl.when(k == 0)
    def _segment_true():
        acc_h_ref[...] = jnp.zeros_like(acc_h_ref)

        rows = jax.lax.broadcasted_iota(jnp.int32, (n, n), 0)
        cols = jax.lax.broadcasted_iota(jnp.int32, (n, n), 1)
        skip_full = skip_ref[...]
        no_skip = jnp.sum(jnp.where(rows == cols, skip_full, 0.0),
                          axis=1, keepdims=True)
        ip = ip_ref[...]
        pbt = ip * no_skip * br_ref[:, 0:1]
        pbf_ref[...] = ip * no_skip * br_ref[:, 1:2]

        ot = (ti_ref[...] == rows).astype(jnp.float32)
        ip_t = jax.lax.dot_general(ot, pbt, dn, precision=hp,
                                   preferred_element_type=jnp.float32)
        out_ip_ref[...] = ip_t
        th = jax.lax.dot_general(ot, h_ref[...] * pbt, dn, precision=hp,
                                 preferred_element_type=jnp.float32)
        out_h_ref[...] = th

    @pl.when(k == 1)
    def _segment_false():
        rows = jax.lax.broadcasted_iota(jnp.int32, (n, n), 0)
        pbf = pbf_ref[...]
        of = (fi_ref[...] == rows).astype(jnp.float32)
        ip_f = jax.lax.dot_general(of, pbf, dn, precision=hp,
                                   preferred_element_type=jnp.float32)
        out_ip_ref[...] += ip_f
        fh = jax.lax.dot_general(of, h_ref[...] * pbf, dn, precision=hp,
                                 preferred_element_type=jnp.float32)
        out_h_ref[...] += fh

    row_ids = k * bi + jax.lax.broadcasted_iota(jnp.int32, (bi, n), 0)
    col_ids = jax.lax.broadcasted_iota(jnp.int32, (bi, n), 1)
    skip_blk = skip_ref[pl.ds(k * bi, bi), :]
    ip_blk = ip_ref[pl.ds(k * bi, bi), :]

    w = jnp.where(row_ids != col_ids, skip_blk, 0.0) * ip_blk

    ones = jnp.ones((bi, 1), jnp.float32)
    out_ip_ref[...] += jax.lax.dot_general(
        w, ones, (((0,), (0,)), ((), ())),
        preferred_element_type=jnp.float32)

    cj = 64
    for jc in range(n // cj):
        js = jc * cj
        hs_c = hs_ref[:, js:js + cj, :]
        w_c = w[:, js:js + cj]
        acc_h_ref[js:js + cj, :] += jnp.sum(hs_c * w_c[:, :, None], axis=0)

    @pl.when(k == nk - 1)
    def _finish():
        new_ip = out_ip_ref[...]
        out_h_ref[...] = (out_h_ref[...] + acc_h_ref[...]) / (new_ip + 1e-7)


@jax.jit
def kernel(instruction_pointer, hidden_state_proposals,
           hidden_state_skip_proposals, skip_decisions, branch_decisions,
           true_indexes, false_indexes):
    n = instruction_pointer.shape[0]
    h_dim = hidden_state_proposals.shape[1]
    block_i = 32
    nk = n // block_i

    ip2 = instruction_pointer.reshape(n, 1)
    ti2 = true_indexes.reshape(1, n)
    fi2 = false_indexes.reshape(1, n)

    out_ip, out_h = pl.pallas_call(
        functools.partial(_body, block_i=block_i),
        grid=(nk,),
        in_specs=[
            pl.BlockSpec((n, 1), lambda k: (0, 0)),
            pl.BlockSpec((n, h_dim), lambda k: (0, 0)),
            pl.BlockSpec((block_i, n, h_dim), lambda k: (k, 0, 0)),
            pl.BlockSpec((n, n), lambda k: (0, 0)),
            pl.BlockSpec((n, 2), lambda k: (0, 0)),
            pl.BlockSpec((1, n), lambda k: (0, 0)),
            pl.BlockSpec((1, n), lambda k: (0, 0)),
        ],
        out_specs=[
            pl.BlockSpec((n, 1), lambda k: (0, 0)),
            pl.BlockSpec((n, h_dim), lambda k: (0, 0)),
        ],
        out_shape=[
            jax.ShapeDtypeStruct((n, 1), jnp.float32),
            jax.ShapeDtypeStruct((n, h_dim), jnp.float32),
        ],
        scratch_shapes=[
            pltpu.VMEM((n, h_dim), jnp.float32),
            pltpu.VMEM((n, 1), jnp.float32),
        ],
    )(ip2, hidden_state_proposals, hidden_state_skip_proposals,
      skip_decisions, branch_decisions, ti2, fi2)

    return out_ip.reshape(n), out_h


# speedup vs baseline: 1.0605x; 1.0605x over previous
import functools

import jax
import jax.numpy as jnp
from jax.experimental import pallas as pl
from jax.experimental.pallas import tpu as pltpu


def _body(ip_ref, h_ref, hs_ref, skip_ref, br_ref, ti_ref, fi_ref,
          out_ip_ref, out_h_ref, acc_h_ref, pbf_ref, *, block_i: int):
    k = pl.program_id(0)
    nk = pl.num_programs(0)
    bi = block_i
    n = skip_ref.shape[1]
    dn = (((1,), (0,)), ((), ()))
    hp = jax.lax.Precision.HIGHEST

    @pl.when(k == 0)
    def _segment_true():
        acc_h_ref[...] = jnp.zeros_like(acc_h_ref)

        rows = jax.lax.broadcasted_iota(jnp.int32, (n, n), 0)
        cols = jax.lax.broadcasted_iota(jnp.int32, (n, n), 1)
        skip_full = skip_ref[...]
        no_skip = jnp.sum(jnp.where(rows == cols, skip_full, 0.0),
                          axis=1, keepdims=True)
        ip = ip_ref[...]
        pbt = ip * no_skip * br_ref[:, 0:1]
        pbf_ref[...] = ip * no_skip * br_ref[:, 1:2]

        ot = (ti_ref[...] == rows).astype(jnp.float32)
        ip_t = jax.lax.dot_general(ot, pbt, dn, precision=hp,
                                   preferred_element_type=jnp.float32)
        out_ip_ref[...] = ip_t
        th = jax.lax.dot_general(ot, h_ref[...] * pbt, dn, precision=hp,
                                 preferred_element_type=jnp.float32)
        out_h_ref[...] = th

    @pl.when(k == 1)
    def _segment_false():
        rows = jax.lax.broadcasted_iota(jnp.int32, (n, n), 0)
        pbf = pbf_ref[...]
        of = (fi_ref[...] == rows).astype(jnp.float32)
        ip_f = jax.lax.dot_general(of, pbf, dn, precision=hp,
                                   preferred_element_type=jnp.float32)
        out_ip_ref[...] += ip_f
        fh = jax.lax.dot_general(of, h_ref[...] * pbf, dn, precision=hp,
                                 preferred_element_type=jnp.float32)
        out_h_ref[...] += fh

    row_ids = k * bi + jax.lax.broadcasted_iota(jnp.int32, (bi, n), 0)
    col_ids = jax.lax.broadcasted_iota(jnp.int32, (bi, n), 1)
    skip_blk = skip_ref[pl.ds(k * bi, bi), :]
    ip_blk = ip_ref[pl.ds(k * bi, bi), :]

    w = jnp.where(row_ids != col_ids, skip_blk, 0.0) * ip_blk

    ones = jnp.ones((bi, 1), jnp.float32)
    out_ip_ref[...] += jax.lax.dot_general(
        w, ones, (((0,), (0,)), ((), ())),
        preferred_element_type=jnp.float32)

    cj = 64
    for jc in range(n // cj):
        js = jc * cj
        hs_c = hs_ref[:, js:js + cj, :]
        w_c = w[:, js:js + cj]
        acc_h_ref[js:js + cj, :] += jnp.sum(hs_c * w_c[:, :, None], axis=0)

    @pl.when(k == nk - 1)
    def _finish():
        new_ip = out_ip_ref[...]
        out_h_ref[...] = (out_h_ref[...] + acc_h_ref[...]) / (new_ip + 1e-7)


@jax.jit
def kernel(instruction_pointer, hidden_state_proposals,
           hidden_state_skip_proposals, skip_decisions, branch_decisions,
           true_indexes, false_indexes):
    n = instruction_pointer.shape[0]
    h_dim = hidden_state_proposals.shape[1]
    block_i = 64
    nk = n // block_i

    ip2 = instruction_pointer.reshape(n, 1)
    ti2 = true_indexes.reshape(1, n)
    fi2 = false_indexes.reshape(1, n)

    out_ip, out_h = pl.pallas_call(
        functools.partial(_body, block_i=block_i),
        grid=(nk,),
        in_specs=[
            pl.BlockSpec((n, 1), lambda k: (0, 0)),
            pl.BlockSpec((n, h_dim), lambda k: (0, 0)),
            pl.BlockSpec((block_i, n, h_dim), lambda k: (k, 0, 0)),
            pl.BlockSpec((n, n), lambda k: (0, 0)),
            pl.BlockSpec((n, 2), lambda k: (0, 0)),
            pl.BlockSpec((1, n), lambda k: (0, 0)),
            pl.BlockSpec((1, n), lambda k: (0, 0)),
        ],
        out_specs=[
            pl.BlockSpec((n, 1), lambda k: (0, 0)),
            pl.BlockSpec((n, h_dim), lambda k: (0, 0)),
        ],
        out_shape=[
            jax.ShapeDtypeStruct((n, 1), jnp.float32),
            jax.ShapeDtypeStruct((n, h_dim), jnp.float32),
        ],
        scratch_shapes=[
            pltpu.VMEM((n, h_dim), jnp.float32),
            pltpu.VMEM((n, 1), jnp.float32),
        ],
    )(ip2, hidden_state_proposals, hidden_state_skip_proposals,
      skip_decisions, branch_decisions, ti2, fi2)

    return out_ip.reshape(n), out_h
